# baseline (speedup 1.0000x reference)
import functools

import jax
import jax.numpy as jnp
from jax import lax
from jax.experimental import pallas as pl
from jax.experimental.pallas import tpu as pltpu
from jax.experimental.pallas import tpu_sc as plsc

N_ROWS_BATCH = 16384
DIM = 32
MEAN = 3.5
LANES = 16

_PERM_DN = lax.GatherDimensionNumbers(
    offset_dims=(), collapsed_slice_dims=(0,), start_index_map=(0,))


def _lane_perm(x, idx):
    return lax.gather(x, idx[:, None], _PERM_DN, slice_sizes=(1,),
                      mode=lax.GatherScatterMode.PROMISE_IN_BOUNDS)


def _sc_body(uidx_hbm, iidx_hbm, ue_hbm, ie_hbm, ub_hbm, ib_hbm, out_hbm,
             uidx_v, iidx_v, ru_v, ri_v, bu_v, bi_v, out_v,
             s0, s1, s2, s3):
    nc = 2
    wid = lax.axis_index("s") * nc + lax.axis_index("c")
    bpw = N_ROWS_BATCH // 32
    base = wid * bpw

    pltpu.sync_copy(uidx_hbm.at[pl.ds(base, bpw)], uidx_v)
    pltpu.sync_copy(iidx_hbm.at[pl.ds(base, bpw)], iidx_v)

    cu = pltpu.async_copy(ue_hbm.at[uidx_v], ru_v, s0)
    ci = pltpu.async_copy(ie_hbm.at[iidx_v], ri_v, s1)
    cbu = pltpu.async_copy(ub_hbm.at[uidx_v], bu_v, s2)
    cbi = pltpu.async_copy(ib_hbm.at[iidx_v], bi_v, s3)
    cu.wait()
    ci.wait()
    cbu.wait()
    cbi.wait()

    lane = lax.iota(jnp.int32, LANES)

    def block(blk, _):
        b0 = blk * LANES
        acc = jnp.zeros((LANES,), jnp.float32)
        for r in range(LANES):
            row = b0 + r
            pu0 = ru_v[row, pl.ds(0, LANES)]
            pu1 = ru_v[row, pl.ds(LANES, LANES)]
            pi0 = ri_v[row, pl.ds(0, LANES)]
            pi1 = ri_v[row, pl.ds(LANES, LANES)]
            prod = pu0 * pi0 + pu1 * pi1
            for sh in (8, 4, 2, 1):
                prod = prod + _lane_perm(prod, lane ^ sh)
            acc = jnp.where(lane == r, prod, acc)
        bias = bu_v[pl.ds(b0, LANES)] + bi_v[pl.ds(b0, LANES)]
        out_v[pl.ds(b0, LANES)] = acc + bias + MEAN
        return _

    lax.fori_loop(0, bpw // LANES, block, 0, unroll=False)
    pltpu.sync_copy(out_v, out_hbm.at[pl.ds(base, bpw)])


@jax.jit
def _sc_rating(user_idx, item_idx, ue, ie, ub, ib):
    bpw = N_ROWS_BATCH // 32
    mesh = plsc.VectorSubcoreMesh(core_axis_name="c", subcore_axis_name="s")
    f = functools.partial(
        pl.kernel,
        mesh=mesh,
        compiler_params=pltpu.CompilerParams(use_tc_tiling_on_sc=False),
        out_type=jax.ShapeDtypeStruct((N_ROWS_BATCH,), jnp.float32),
        scratch_types=[
            pltpu.VMEM((bpw,), jnp.int32),
            pltpu.VMEM((bpw,), jnp.int32),
            pltpu.VMEM((bpw, DIM), jnp.float32),
            pltpu.VMEM((bpw, DIM), jnp.float32),
            pltpu.VMEM((bpw,), jnp.float32),
            pltpu.VMEM((bpw,), jnp.float32),
            pltpu.VMEM((bpw,), jnp.float32),
            pltpu.SemaphoreType.DMA,
            pltpu.SemaphoreType.DMA,
            pltpu.SemaphoreType.DMA,
            pltpu.SemaphoreType.DMA,
        ],
    )(_sc_body)
    return f(user_idx, item_idx, ue, ie, ub, ib)


def kernel(inputs, user_embedding, item_embedding, user_bias, item_bias):
    user_idx = inputs[:, 0]
    item_idx = inputs[:, 1]
    rating = _sc_rating(user_idx, item_idx,
                        user_embedding, item_embedding,
                        user_bias.reshape(-1), item_bias.reshape(-1))
    return rating.reshape(N_ROWS_BATCH, 1)
